# baseline (speedup 1.0000x reference)
import functools

import jax
import jax.numpy as jnp
from jax import lax
from jax.experimental import pallas as pl
from jax.experimental.pallas import tpu as pltpu
from jax.experimental.pallas import tpu_sc as plsc

D = 64
NC = 2
NS = 16
NW = NC * NS
CHUNK = 128
NBUF = 4


def _build_gather(B: int):
    b_per_w = B // NW
    n_chunks = b_per_w // CHUNK
    assert b_per_w % CHUNK == 0 and (n_chunks - NBUF) % NBUF == 0

    mesh = plsc.VectorSubcoreMesh(core_axis_name="c", subcore_axis_name="s")

    @functools.partial(
        pl.kernel,
        mesh=mesh,
        out_type=jax.ShapeDtypeStruct((B, D), jnp.float32),
        compiler_params=pltpu.CompilerParams(use_tc_tiling_on_sc=False),
        scratch_types=[
            pltpu.VMEM((b_per_w,), jnp.int32),
            pltpu.VMEM((NBUF, CHUNK, D), jnp.float32),
        ] + [pltpu.SemaphoreType.DMA] * NBUF,
    )
    def gather_kernel(table_hbm, idx_hbm, out_hbm, idx_v, rows_v, *sems):
        wid = lax.axis_index("s") * NC + lax.axis_index("c")
        base = wid * b_per_w
        pltpu.sync_copy(idx_hbm.at[pl.ds(base, b_per_w)], idx_v)

        def issue(chunk, buf):
            idx_slice = idx_v.at[pl.ds(chunk * CHUNK, CHUNK)]
            pltpu.async_copy(table_hbm.at[idx_slice], rows_v.at[buf], sems[buf])

        def wait_and_flush(chunk, buf):
            idx_slice = idx_v.at[pl.ds(chunk * CHUNK, CHUNK)]
            pltpu.make_async_copy(
                table_hbm.at[idx_slice], rows_v.at[buf], sems[buf]
            ).wait()
            pltpu.sync_copy(
                rows_v.at[buf], out_hbm.at[pl.ds(base + chunk * CHUNK, CHUNK)]
            )

        for b in range(NBUF):
            issue(b, b)

        @pl.loop(0, n_chunks - NBUF, step=NBUF)
        def _(g0):
            for b in range(NBUF):
                g = g0 + b
                wait_and_flush(g, b)
                issue(g + NBUF, b)

        for b in range(NBUF):
            wait_and_flush(n_chunks - NBUF + b, b)

    return gather_kernel


def kernel(x, weight):
    B = x.size
    idx = x.reshape(B).astype(jnp.int32)
    out = _build_gather(B)(weight, idx)
    return out.reshape(x.shape + (D,))
